# baseline (speedup 1.0000x reference)
import functools

import jax
import jax.numpy as jnp
from jax import lax
from jax.experimental import pallas as pl
from jax.experimental.pallas import tpu as pltpu
from jax.experimental.pallas import tpu_sc as plsc

D_MODEL = 64
SCALE = 8.0

NUM_CORES = 2
NUM_SUBCORES = 16
NUM_WORKERS = NUM_CORES * NUM_SUBCORES

BBLK = 128
NBUF = 2


@functools.lru_cache(maxsize=None)
def _build(S, B):
    assert B == NUM_WORKERS * BBLK
    assert S % NBUF == 0

    mesh = plsc.VectorSubcoreMesh(
        core_axis_name="c",
        subcore_axis_name="s",
        num_cores=NUM_CORES,
        num_subcores=NUM_SUBCORES,
    )

    @functools.partial(
        pl.kernel,
        out_type=jax.ShapeDtypeStruct((S, 8, NUM_WORKERS, 8 * BBLK), jnp.float32),
        mesh=mesh,
        scratch_types=[
            pltpu.VMEM((S, BBLK), jnp.int32),
            [pltpu.VMEM((BBLK, D_MODEL), jnp.float32) for _ in range(NBUF)],
            [pltpu.VMEM((8, 8 * BBLK), jnp.float32) for _ in range(NBUF)],
            [pltpu.SemaphoreType.DMA for _ in range(NBUF)],
            [pltpu.SemaphoreType.DMA for _ in range(NBUF)],
        ],
        compiler_params=pltpu.CompilerParams(
            use_tc_tiling_on_sc=False, needs_layout_passes=False
        ),
    )
    def emb_kernel(xt_hbm, table_hbm, out_hbm, idx_v, gbufs, obufs, gsems, osems):
        w = lax.axis_index("s") * NUM_CORES + lax.axis_index("c")
        pltpu.sync_copy(xt_hbm.at[:, pl.ds(w * BBLK, BBLK)], idx_v)

        def gather(s, b):
            return pltpu.make_async_copy(
                table_hbm.at[idx_v.at[s]], gbufs[b], gsems[b]
            )

        def out_write(s, b):
            return pltpu.make_async_copy(obufs[b], out_hbm.at[s, :, w], osems[b])

        for b in range(NBUF):
            gather(b, b).start()

        iota = lax.iota(jnp.int32, 16)

        @pl.loop(0, S, step=NBUF)
        def _group(g):
            for b in range(NBUF):
                s = g + b
                gather(s, b).wait()

                @pl.when(s >= NBUF)
                def _():
                    out_write(s - NBUF, b).wait()

                @pl.loop(0, 8)
                def _tc(t_c):
                    for r in range(8):
                        col = t_c * 8 + r
                        colv = jnp.full((16,), 0, jnp.int32) + col
                        for k in range(8):
                            rows = iota + (16 * k)
                            vals = plsc.load_gather(gbufs[b], [rows, colv])
                            obufs[b][t_c, pl.ds(r * BBLK + 16 * k, 16)] = (
                                vals * SCALE
                            )

                @pl.when(s + NBUF < S)
                def _():
                    gather(s + NBUF, b).start()

                out_write(s, b).start()

        for b in range(NBUF):
            out_write(S - NBUF + b, b).wait()

    return emb_kernel


def kernel(x, lut):
    bsz, seq = x.shape
    xt = jnp.transpose(x)
    out5 = _build(seq, bsz)(xt, lut)
    out = (
        out5.reshape(seq, 8, bsz // BBLK, 8, BBLK)
        .transpose(2, 4, 0, 1, 3)
        .reshape(bsz, seq, D_MODEL)
    )
    return out


# speedup vs baseline: 1.7854x; 1.7854x over previous
import functools

import jax
import jax.numpy as jnp
from jax import lax
from jax.experimental import pallas as pl
from jax.experimental.pallas import tpu as pltpu
from jax.experimental.pallas import tpu_sc as plsc

D_MODEL = 64
SCALE = 8.0

NUM_CORES = 2
NUM_SUBCORES = 16
NUM_WORKERS = NUM_CORES * NUM_SUBCORES

BBLK = 128
PITCH = BBLK + 1
NBUF = 2


@functools.lru_cache(maxsize=None)
def _build(S, B):
    assert B == NUM_WORKERS * BBLK
    assert S % NBUF == 0

    mesh = plsc.VectorSubcoreMesh(
        core_axis_name="c",
        subcore_axis_name="s",
        num_cores=NUM_CORES,
        num_subcores=NUM_SUBCORES,
    )

    @functools.partial(
        pl.kernel,
        out_type=jax.ShapeDtypeStruct((S, 8, NUM_WORKERS, 8, BBLK), jnp.float32),
        mesh=mesh,
        scratch_types=[
            pltpu.VMEM((S, BBLK), jnp.int32),
            [pltpu.VMEM((BBLK, D_MODEL), jnp.float32) for _ in range(NBUF)],
            [pltpu.VMEM((8, 8, PITCH), jnp.float32) for _ in range(NBUF)],
            [pltpu.SemaphoreType.DMA for _ in range(NBUF)],
            [pltpu.SemaphoreType.DMA for _ in range(NBUF)],
        ],
        compiler_params=pltpu.CompilerParams(
            use_tc_tiling_on_sc=False, needs_layout_passes=False
        ),
    )
    def emb_kernel(xt_hbm, table_hbm, out_hbm, idx_v, gbufs, obufs, gsems, osems):
        w = lax.axis_index("s") * NUM_CORES + lax.axis_index("c")
        pltpu.sync_copy(xt_hbm.at[:, pl.ds(w * BBLK, BBLK)], idx_v)

        def gather(s, b):
            return pltpu.make_async_copy(
                table_hbm.at[idx_v.at[s]], gbufs[b], gsems[b]
            )

        def out_write(s, b):
            return pltpu.make_async_copy(
                obufs[b].at[:, :, pl.ds(0, BBLK)],
                out_hbm.at[s, :, w],
                osems[b],
            )

        for b in range(NBUF):
            gather(b, b).start()

        iota = lax.iota(jnp.int32, 16)
        tcs = [(iota + 16 * m) // 8 for m in range(D_MODEL // 16)]
        rs = [(iota + 16 * m) % 8 for m in range(D_MODEL // 16)]

        @pl.loop(0, S, step=NBUF)
        def _group(g):
            for b in range(NBUF):
                s = g + b
                gather(s, b).wait()

                @pl.when(s >= NBUF)
                def _():
                    out_write(s - NBUF, b).wait()

                @pl.loop(0, BBLK, unroll=2)
                def _row(l):
                    lv = jnp.full((16,), 0, jnp.int32) + l
                    for m in range(D_MODEL // 16):
                        vals = gbufs[b][l, pl.ds(16 * m, 16)] * SCALE
                        plsc.store_scatter(obufs[b], [tcs[m], rs[m], lv], vals)

                @pl.when(s + NBUF < S)
                def _():
                    gather(s + NBUF, b).start()

                out_write(s, b).start()

        for b in range(NBUF):
            out_write(S - NBUF + b, b).wait()

    return emb_kernel


def kernel(x, lut):
    bsz, seq = x.shape
    xt = jnp.transpose(x)
    out5 = _build(seq, bsz)(xt, lut)
    out = out5.transpose(2, 4, 0, 1, 3).reshape(bsz, seq, D_MODEL)
    return out
